# baseline (speedup 1.0000x reference)
import functools

import jax
import jax.numpy as jnp
from jax import lax
from jax.experimental import pallas as pl
from jax.experimental.pallas import tpu as pltpu
from jax.experimental.pallas import tpu_sc as plsc

_NC = 2
_NS = 16
_NW = _NC * _NS
_CHUNK = 128
_PACK = 4


def _sc_gather3(ubk, ibk, ue4, ur4, ie4):
    n_chunks = ubk.shape[1]
    b_per_w = n_chunks * _CHUNK
    batch = _NW * b_per_w
    mesh = plsc.VectorSubcoreMesh(
        core_axis_name="c", subcore_axis_name="s",
        num_cores=_NC, num_subcores=_NS)
    out_t = jax.ShapeDtypeStruct((batch, _CHUNK), jnp.float32)

    @functools.partial(
        pl.kernel,
        out_type=(out_t, out_t, out_t),
        mesh=mesh,
        scratch_types=[
            pltpu.VMEM((n_chunks, _CHUNK), jnp.int32),
            pltpu.VMEM((n_chunks, _CHUNK), jnp.int32),
            pltpu.VMEM((b_per_w, _CHUNK), jnp.float32),
            pltpu.SemaphoreType.DMA,
            pltpu.SemaphoreType.DMA,
        ],
    )
    def k(ue_t, ur_t, ie_t, ubk_h, ibk_h, ue_o, ur_o, ie_o,
          ubk_v, ibk_v, rows, gsem, ssem):
        wid = lax.axis_index("s") * _NC + lax.axis_index("c")
        pltpu.sync_copy(ubk_h.at[wid], ubk_v)
        pltpu.sync_copy(ibk_h.at[wid], ibk_v)

        for tbl, idxv, out in ((ue_t, ubk_v, ue_o),
                               (ur_t, ubk_v, ur_o),
                               (ie_t, ibk_v, ie_o)):
            gathers = [pltpu.async_copy(
                tbl.at[idxv.at[c]],
                rows.at[pl.ds(c * _CHUNK, _CHUNK)], gsem)
                for c in range(n_chunks)]
            stores = []
            for c, g in enumerate(gathers):
                g.wait()
                stores.append(pltpu.async_copy(
                    rows.at[pl.ds(c * _CHUNK, _CHUNK)],
                    out.at[pl.ds(wid * b_per_w + c * _CHUNK, _CHUNK)],
                    ssem))
            for s in stores:
                s.wait()

    return k(ue4, ur4, ie4, ubk, ibk)


def _tc_body(ue_ref, ur_ref, ie_ref, uix_ref, iix_ref, c_ref, w_ref, b_ref,
             o_ref, *, num_clusters):
    dim = c_ref.shape[1]
    uoff = jnp.bitwise_and(uix_ref[...], _PACK - 1)
    ioff = jnp.bitwise_and(iix_ref[...], _PACK - 1)

    def sel(x4, off):
        acc = jnp.where(off == 0, 1.0, 0.0) * x4[:, 0:dim]
        for kk in range(1, _PACK):
            acc += jnp.where(off == kk, 1.0, 0.0) * x4[:, kk * dim:(kk + 1) * dim]
        return acc

    rep = sel(ur_ref[...], uoff)
    c = c_ref[...]
    cp = c.shape[0]
    scores = -2.0 * lax.dot_general(rep, c, (((1,), (1,)), ((), ())),
                                    preferred_element_type=jnp.float32)
    cn = jnp.sum(c * c, axis=1)
    pad = jnp.where(lax.broadcasted_iota(jnp.int32, (cp,), 0) < num_clusters,
                    0.0, jnp.float32(1e30))
    scores = scores + (cn + pad)[None, :]
    nearest = jnp.argmin(scores, axis=1)
    onehot = (lax.broadcasted_iota(jnp.int32, scores.shape, 1)
              == nearest[:, None]).astype(jnp.float32)
    proto = lax.dot_general(onehot, c, (((1,), (0,)), ((), ())),
                            preferred_element_type=jnp.float32)
    prod = sel(ue_ref[...], uoff) * proto * sel(ie_ref[...], ioff)
    logit = jnp.sum(prod * w_ref[...], axis=1, keepdims=True) + b_ref[0]
    o_ref[...] = jax.nn.sigmoid(logit)


def _tc_tail(ue4, ur4, ie4, uix, iix, centers, W, b, *, num_clusters,
             interpret=False):
    batch = ue4.shape[0]
    dim = centers.shape[1]
    cp = 128
    c_pad = jnp.zeros((cp, dim), centers.dtype).at[:num_clusters].set(centers)
    blk = 2048
    grid = (batch // blk,)
    row_spec = pl.BlockSpec((blk, _CHUNK), lambda i: (i, 0))
    ix_spec = pl.BlockSpec((blk, 1), lambda i: (i, 0))
    return pl.pallas_call(
        functools.partial(_tc_body, num_clusters=num_clusters),
        grid=grid,
        in_specs=[
            row_spec, row_spec, row_spec,
            ix_spec, ix_spec,
            pl.BlockSpec((cp, dim), lambda i: (0, 0)),
            pl.BlockSpec((1, dim), lambda i: (0, 0)),
            pl.BlockSpec(memory_space=pltpu.SMEM),
        ],
        out_specs=pl.BlockSpec((blk, 1), lambda i: (i, 0)),
        out_shape=jax.ShapeDtypeStruct((batch, 1), jnp.float32),
        interpret=interpret,
    )(ue4, ur4, ie4, uix, iix, c_pad, W, b)


def kernel(user_indices, item_indices, emb_user, emb_item, user_reprs,
           cluster_centers, W, b):
    num_clusters = cluster_centers.shape[0]
    batch = user_indices.shape[0]
    n_rows, dim = emb_user.shape
    n_chunks = batch // (_NW * _CHUNK)
    uix = user_indices.astype(jnp.int32)
    iix = item_indices.astype(jnp.int32)
    ubk = (uix >> 2).reshape(_NW, n_chunks, _CHUNK)
    ibk = (iix >> 2).reshape(_NW, n_chunks, _CHUNK)
    ue4 = emb_user.reshape(n_rows // _PACK, _PACK * dim)
    ur4 = user_reprs.reshape(n_rows // _PACK, _PACK * dim)
    ie4 = emb_item.reshape(emb_item.shape[0] // _PACK, _PACK * dim)
    ue_g, ur_g, ie_g = _sc_gather3(ubk, ibk, ue4, ur4, ie4)
    return _tc_tail(ue_g, ur_g, ie_g, uix.reshape(batch, 1),
                    iix.reshape(batch, 1), cluster_centers, W, b,
                    num_clusters=num_clusters)


# speedup vs baseline: 1.4751x; 1.4751x over previous
import functools

import jax
import jax.numpy as jnp
from jax import lax
from jax.experimental import pallas as pl
from jax.experimental.pallas import tpu as pltpu
from jax.experimental.pallas import tpu_sc as plsc

_NC = 2
_NS = 16
_NW = _NC * _NS


def _sc_gather1(table, indices):
    batch, dim = indices.shape[0], table.shape[1]
    b_per_w = batch // _NW
    mesh = plsc.VectorSubcoreMesh(
        core_axis_name="c", subcore_axis_name="s",
        num_cores=_NC, num_subcores=_NS)
    out_t = jax.ShapeDtypeStruct((batch, dim), jnp.float32)

    @functools.partial(
        pl.kernel,
        out_type=out_t,
        mesh=mesh,
        compiler_params=pltpu.CompilerParams(needs_layout_passes=False),
        scratch_types=[
            pltpu.VMEM((b_per_w,), jnp.int32),
            pltpu.VMEM((b_per_w, dim), jnp.float32),
            pltpu.SemaphoreType.DMA,
            pltpu.SemaphoreType.DMA,
        ],
    )
    def k(tbl_hbm, idx_hbm, out_hbm, idx_v, rows, gsem, ssem):
        wid = lax.axis_index("s") * _NC + lax.axis_index("c")
        base = wid * b_per_w
        sl = pl.ds(base, b_per_w)
        pltpu.sync_copy(idx_hbm.at[sl], idx_v)
        lanes = lax.broadcasted_iota(jnp.int32, (16,), 0)

        @pl.loop(0, b_per_w)
        def _(j):
            v16 = (j // 16) * 16
            vec = idx_v[pl.ds(v16, 16)]
            i = jnp.sum(jnp.where(lanes == (j - v16), vec, jnp.int32(0)))
            pltpu.async_copy(tbl_hbm.at[i], rows.at[j], gsem)

        pltpu.make_async_copy(tbl_hbm.at[pl.ds(0, b_per_w)], rows, gsem).wait()
        pltpu.async_copy(rows, out_hbm.at[sl], ssem).wait()

    return k(table, indices)


def _sc_gather3(user_indices, item_indices, emb_user, emb_item, user_reprs):
    uix = user_indices.astype(jnp.int32)
    iix = item_indices.astype(jnp.int32)
    ue = _sc_gather1(emb_user, uix)
    ur = _sc_gather1(user_reprs, uix)
    ie = _sc_gather1(emb_item, iix)
    return ue, ur, ie


def _tc_body(ue_ref, ur_ref, ie_ref, c_ref, w_ref, b_ref, o_ref,
             *, num_clusters):
    rep = ur_ref[...]
    c = c_ref[...]
    cp = c.shape[0]
    scores = -2.0 * lax.dot_general(rep, c, (((1,), (1,)), ((), ())),
                                    preferred_element_type=jnp.float32)
    cn = jnp.sum(c * c, axis=1)
    pad = jnp.where(lax.broadcasted_iota(jnp.int32, (cp,), 0) < num_clusters,
                    0.0, jnp.float32(1e30))
    scores = scores + (cn + pad)[None, :]
    nearest = jnp.argmin(scores, axis=1)
    onehot = (lax.broadcasted_iota(jnp.int32, scores.shape, 1)
              == nearest[:, None]).astype(jnp.float32)
    proto = lax.dot_general(onehot, c, (((1,), (0,)), ((), ())),
                            preferred_element_type=jnp.float32)
    prod = ue_ref[...] * proto * ie_ref[...]
    logit = jnp.sum(prod * w_ref[...], axis=1, keepdims=True) + b_ref[0]
    o_ref[...] = jax.nn.sigmoid(logit)


def _tc_tail(ue, ur, ie, centers, W, b, *, num_clusters, interpret=False):
    batch, dim = ue.shape
    cp = 128
    c_pad = jnp.zeros((cp, dim), centers.dtype).at[:num_clusters].set(centers)
    blk = 2048
    grid = (batch // blk,)
    row_spec = pl.BlockSpec((blk, dim), lambda i: (i, 0))
    return pl.pallas_call(
        functools.partial(_tc_body, num_clusters=num_clusters),
        grid=grid,
        in_specs=[
            row_spec, row_spec, row_spec,
            pl.BlockSpec((cp, dim), lambda i: (0, 0)),
            pl.BlockSpec((1, dim), lambda i: (0, 0)),
            pl.BlockSpec(memory_space=pltpu.SMEM),
        ],
        out_specs=pl.BlockSpec((blk, 1), lambda i: (i, 0)),
        out_shape=jax.ShapeDtypeStruct((batch, 1), jnp.float32),
        interpret=interpret,
    )(ue, ur, ie, c_pad, W, b)


def kernel(user_indices, item_indices, emb_user, emb_item, user_reprs,
           cluster_centers, W, b):
    num_clusters = cluster_centers.shape[0]
    ue, ur, ie = _sc_gather3(user_indices, item_indices,
                             emb_user, emb_item, user_reprs)
    return _tc_tail(ue, ur, ie, cluster_centers, W, b,
                    num_clusters=num_clusters)
